# baseline (speedup 1.0000x reference)
import functools

import numpy as np
import jax
import jax.numpy as jnp
from jax import lax
from jax.experimental import pallas as pl
from jax.experimental.pallas import tpu as pltpu
from jax.experimental.pallas import tpu_sc as plsc

_HIDDEN = 128
_MAXLEN = 60
_EPS = 1e-5


def _pe_const(seq_len: int) -> jnp.ndarray:
    position = np.arange(0, _MAXLEN, dtype=np.float32)[:, None]
    div_term = 1.0 / (
        10000.0 ** (np.arange(0, _HIDDEN, 2, dtype=np.float32) * 2.0 / _HIDDEN)
    )
    pe = np.zeros((_MAXLEN, _HIDDEN), dtype=np.float32)
    pe[:, 0::2] = np.sin(position * div_term)
    pe[:, 1::2] = np.cos(position * div_term)
    return jnp.asarray(pe[:seq_len])


def _sc_gather(table: jnp.ndarray, idx: jnp.ndarray) -> jnp.ndarray:
    b = idx.shape[0]
    h = table.shape[1]
    window = 128
    idx2 = idx.reshape(1, b)
    mesh = plsc.VectorSubcoreMesh(
        core_axis_name="core", subcore_axis_name="subcore"
    )

    @functools.partial(
        pl.kernel,
        out_type=jax.ShapeDtypeStruct((b, h), table.dtype),
        mesh=mesh,
    )
    def gather_kernel(tab_hbm, i_hbm, o_hbm):
        def body(i_vmem, o_vmem):
            pltpu.sync_copy(tab_hbm.at[i_vmem.at[0]], o_vmem)

        pltpu.emit_pipeline(
            body,
            grid=(b // window,),
            in_specs=[pl.BlockSpec((1, window), index_map=lambda i: (0, i))],
            out_specs=[pl.BlockSpec((window, h), index_map=lambda i: (i, 0))],
            core_axis_name=("core", "subcore"),
            dimension_semantics=(pltpu.PARALLEL,),
        )(i_hbm, o_hbm)

    return gather_kernel(table, idx2)


def _stats_outer(cs, b, nsteps, nbuf, x_hbm, pe_hbm, o_ref, sbh_ref):
    o_ref[...] = jnp.zeros((8, _HIDDEN), jnp.float32)
    sbh_ref[...] = jnp.zeros((b, _HIDDEN), jnp.float32)

    def chunk_body(xa_vmem, xb_vmem, pa_vmem, pb_vmem):
        acc8 = o_ref[...]
        accbh = sbh_ref[...]
        for x_vmem, pe_vmem in ((xa_vmem, pa_vmem), (xb_vmem, pb_vmem)):
            x3 = x_vmem[...].reshape(cs, b, _HIDDEN)
            colsum = jnp.sum(x3, axis=1)
            s = jnp.sum(colsum, axis=0).reshape(1, _HIDDEN)
            ss = jnp.sum(x3 * x3, axis=(0, 1)).reshape(1, _HIDDEN)
            pex = jnp.sum(pe_vmem[:, 0, :] * colsum, axis=0).reshape(
                1, _HIDDEN
            )
            acc8 = acc8 + jnp.concatenate(
                [s, ss, pex, jnp.zeros((5, _HIDDEN), jnp.float32)], axis=0
            )
            accbh = accbh + jnp.sum(x3, axis=0)
        o_ref[...] = acc8
        sbh_ref[...] = accbh

    half = nsteps // 2
    pltpu.emit_pipeline(
        chunk_body,
        grid=(half,),
        in_specs=[
            pl.BlockSpec(
                (cs * b, _HIDDEN),
                lambda i: (i, 0),
                pipeline_mode=pl.Buffered(buffer_count=nbuf),
            ),
            pl.BlockSpec(
                (cs * b, _HIDDEN),
                lambda i: (i + half, 0),
                pipeline_mode=pl.Buffered(buffer_count=nbuf),
            ),
            pl.BlockSpec((cs, 1, _HIDDEN), lambda i: (i, 0, 0)),
            pl.BlockSpec((cs, 1, _HIDDEN), lambda i: (i + half, 0, 0)),
        ],
    )(x_hbm, x_hbm, pe_hbm, pe_hbm)


def _norm_body(n_total, seq_len, n_batch, x_ref, emb_ref, pe_ref, st_ref,
               sbh_ref, pc_ref, w_ref, b_ref, o_ref, sc_ref):
    i = pl.program_id(0)

    @pl.when(i == 0)
    def _derive():
        inv_n = 1.0 / n_total
        st = st_ref[...]
        spe = pc_ref[0:1, :]
        spe2 = pc_ref[1:2, :]
        e = emb_ref[...]
        se = jnp.sum(e, axis=0).reshape(1, _HIDDEN)
        se2 = jnp.sum(e * e, axis=0).reshape(1, _HIDDEN)
        cross_x = jnp.sum(e * sbh_ref[...], axis=0).reshape(1, _HIDDEN)
        sum_y = st[0:1, :] + n_batch * spe + seq_len * se
        ssq_y = (
            st[1:2, :]
            + 2.0 * st[2:3, :]
            + n_batch * spe2
            + seq_len * se2
            + 2.0 * (cross_x + se * spe)
        )
        mean = sum_y * inv_n
        var = ssq_y * inv_n - mean * mean
        scale = w_ref[...] * lax.rsqrt(var + _EPS)
        shift = b_ref[...] - mean * scale
        sc_ref[...] = jnp.concatenate(
            [scale, shift, jnp.zeros((6, _HIDDEN), jnp.float32)], axis=0
        )

    scale = sc_ref[0:1, :]
    shift = sc_ref[1:2, :]
    k = pe_ref.shape[0]
    b = emb_ref.shape[0]
    x3 = x_ref[...].reshape(k, b, _HIDDEN)
    y = x3 + emb_ref[...][None] + pe_ref[...]
    o_ref[...] = (y * scale + shift).reshape(k * b, _HIDDEN)


def kernel(x, g_id, emb_table, bn_weight, bn_bias):
    b, l, h = x.shape
    pe = _pe_const(l)
    emb = _sc_gather(emb_table, g_id)

    x2 = jnp.transpose(x, (1, 0, 2)).reshape(l * b, h)

    pe3 = pe.reshape(l, 1, h)
    ks = 1
    kn = 2
    emb_spec = pl.BlockSpec((b, h), lambda i: (0, 0))

    stats, sbh = pl.pallas_call(
        functools.partial(_stats_outer, ks, b, l // ks, 8),
        in_specs=[
            pl.BlockSpec(memory_space=pltpu.MemorySpace.HBM),
            pl.BlockSpec(memory_space=pltpu.MemorySpace.HBM),
        ],
        out_shape=[
            jax.ShapeDtypeStruct((8, h), jnp.float32),
            jax.ShapeDtypeStruct((b, h), jnp.float32),
        ],
    )(x2, pe3)

    w2 = bn_weight.reshape(1, h)
    b2 = bn_bias.reshape(1, h)
    pe_consts = jnp.concatenate(
        [
            jnp.sum(pe, axis=0).reshape(1, h),
            jnp.sum(pe * pe, axis=0).reshape(1, h),
            jnp.zeros((6, h), jnp.float32),
        ],
        axis=0,
    )
    row_spec = pl.BlockSpec((1, h), lambda i: (0, 0))
    st8_spec = pl.BlockSpec((8, h), lambda i: (0, 0))
    nblk_spec = pl.BlockSpec((kn * b, h), lambda i: (i, 0))

    out2 = pl.pallas_call(
        functools.partial(_norm_body, float(b * l), float(l), float(b)),
        grid=(l // kn,),
        in_specs=[
            nblk_spec,
            emb_spec,
            pl.BlockSpec((kn, 1, h), lambda i: (i, 0, 0)),
            st8_spec,
            emb_spec,
            st8_spec,
            row_spec,
            row_spec,
        ],
        out_specs=nblk_spec,
        out_shape=jax.ShapeDtypeStruct((l * b, h), jnp.float32),
        scratch_shapes=[pltpu.VMEM((8, h), jnp.float32)],
        compiler_params=pltpu.CompilerParams(
            dimension_semantics=("arbitrary",)
        ),
    )(x2, emb, pe3, stats, sbh, pe_consts, w2, b2)

    return jnp.transpose(out2.reshape(l, b, h), (1, 0, 2))


# speedup vs baseline: 1.0233x; 1.0233x over previous
import functools

import numpy as np
import jax
import jax.numpy as jnp
from jax import lax
from jax.experimental import pallas as pl
from jax.experimental.pallas import tpu as pltpu
from jax.experimental.pallas import tpu_sc as plsc

_HIDDEN = 128
_MAXLEN = 60
_EPS = 1e-5


def _pe_const(seq_len: int) -> jnp.ndarray:
    position = np.arange(0, _MAXLEN, dtype=np.float32)[:, None]
    div_term = 1.0 / (
        10000.0 ** (np.arange(0, _HIDDEN, 2, dtype=np.float32) * 2.0 / _HIDDEN)
    )
    pe = np.zeros((_MAXLEN, _HIDDEN), dtype=np.float32)
    pe[:, 0::2] = np.sin(position * div_term)
    pe[:, 1::2] = np.cos(position * div_term)
    return jnp.asarray(pe[:seq_len])


def _sc_gather(table: jnp.ndarray, idx: jnp.ndarray) -> jnp.ndarray:
    b = idx.shape[0]
    h = table.shape[1]
    window = 128
    idx2 = idx.reshape(1, b)
    mesh = plsc.VectorSubcoreMesh(
        core_axis_name="core", subcore_axis_name="subcore"
    )

    @functools.partial(
        pl.kernel,
        out_type=jax.ShapeDtypeStruct((b, h), table.dtype),
        mesh=mesh,
    )
    def gather_kernel(tab_hbm, i_hbm, o_hbm):
        def body(i_vmem, o_vmem):
            pltpu.sync_copy(tab_hbm.at[i_vmem.at[0]], o_vmem)

        pltpu.emit_pipeline(
            body,
            grid=(b // window,),
            in_specs=[pl.BlockSpec((1, window), index_map=lambda i: (0, i))],
            out_specs=[pl.BlockSpec((window, h), index_map=lambda i: (i, 0))],
            core_axis_name=("core", "subcore"),
            dimension_semantics=(pltpu.PARALLEL,),
        )(i_hbm, o_hbm)

    return gather_kernel(table, idx2)


def _stats_body(k, b, x_ref, pe_ref, o_ref, sbh_ref, acc_ref):
    i = pl.program_id(0)
    n = pl.num_programs(0)
    x3 = x_ref[...].reshape(k, b, _HIDDEN)
    colsum = jnp.sum(x3, axis=1)
    s = jnp.sum(colsum, axis=0).reshape(1, _HIDDEN)
    ss = jnp.sum(x3 * x3, axis=(0, 1)).reshape(1, _HIDDEN)
    pex = jnp.sum(pe_ref[:, 0, :] * colsum, axis=0).reshape(1, _HIDDEN)
    o_ref[...] = jnp.concatenate(
        [s, ss, pex, jnp.zeros((5, _HIDDEN), jnp.float32)], axis=0
    )[None]
    part = jnp.sum(x3, axis=0)

    @pl.when(i == 0)
    def _init():
        acc_ref[...] = part

    @pl.when(i != 0)
    def _acc():
        acc_ref[...] = acc_ref[...] + part

    @pl.when(i == n - 1)
    def _flush():
        sbh_ref[...] = acc_ref[...]


def _norm_body(n_total, seq_len, n_batch, x_ref, emb_ref, pe_ref, st_ref,
               sbh_ref, pc_ref, w_ref, b_ref, o_ref, sc_ref):
    i = pl.program_id(0)

    @pl.when(i == 0)
    def _derive():
        inv_n = 1.0 / n_total
        st = jnp.sum(st_ref[...], axis=0)
        spe = pc_ref[0:1, :]
        spe2 = pc_ref[1:2, :]
        e = emb_ref[...]
        se = jnp.sum(e, axis=0).reshape(1, _HIDDEN)
        se2 = jnp.sum(e * e, axis=0).reshape(1, _HIDDEN)
        cross_x = jnp.sum(e * sbh_ref[...], axis=0).reshape(1, _HIDDEN)
        sum_y = st[0:1, :] + n_batch * spe + seq_len * se
        ssq_y = (
            st[1:2, :]
            + 2.0 * st[2:3, :]
            + n_batch * spe2
            + seq_len * se2
            + 2.0 * (cross_x + se * spe)
        )
        mean = sum_y * inv_n
        var = ssq_y * inv_n - mean * mean
        scale = w_ref[...] * lax.rsqrt(var + _EPS)
        shift = b_ref[...] - mean * scale
        sc_ref[...] = jnp.concatenate(
            [scale, shift, jnp.zeros((6, _HIDDEN), jnp.float32)], axis=0
        )

    scale = sc_ref[0:1, :]
    shift = sc_ref[1:2, :]
    k = pe_ref.shape[0]
    b = emb_ref.shape[0]
    x3 = x_ref[...].reshape(k, b, _HIDDEN)
    y = x3 + emb_ref[...][None] + pe_ref[...]
    o_ref[...] = (y * scale + shift).reshape(k * b, _HIDDEN)


def kernel(x, g_id, emb_table, bn_weight, bn_bias):
    b, l, h = x.shape
    pe = _pe_const(l)
    emb = _sc_gather(emb_table, g_id)

    x2 = jnp.transpose(x, (1, 0, 2)).reshape(l * b, h)

    pe3 = pe.reshape(l, 1, h)
    ks = 5
    kn = 2
    emb_spec = pl.BlockSpec((b, h), lambda i: (0, 0))

    stats8, sbh = pl.pallas_call(
        functools.partial(_stats_body, ks, b),
        grid=(l // ks,),
        in_specs=[
            pl.BlockSpec((ks * b, h), lambda i: (i, 0)),
            pl.BlockSpec((ks, 1, h), lambda i: (i, 0, 0)),
        ],
        out_specs=[
            pl.BlockSpec((1, 8, h), lambda i: (i, 0, 0)),
            pl.BlockSpec((b, h), lambda i: (0, 0)),
        ],
        out_shape=[
            jax.ShapeDtypeStruct((l // ks, 8, h), jnp.float32),
            jax.ShapeDtypeStruct((b, h), jnp.float32),
        ],
        scratch_shapes=[pltpu.VMEM((b, h), jnp.float32)],
        compiler_params=pltpu.CompilerParams(
            dimension_semantics=("arbitrary",)
        ),
    )(x2, pe3)

    w2 = bn_weight.reshape(1, h)
    b2 = bn_bias.reshape(1, h)
    pe_consts = jnp.concatenate(
        [
            jnp.sum(pe, axis=0).reshape(1, h),
            jnp.sum(pe * pe, axis=0).reshape(1, h),
            jnp.zeros((6, h), jnp.float32),
        ],
        axis=0,
    )
    row_spec = pl.BlockSpec((1, h), lambda i: (0, 0))
    st8_spec = pl.BlockSpec((8, h), lambda i: (0, 0))
    st_spec = pl.BlockSpec((l // ks, 8, h), lambda i: (0, 0, 0))
    nblk_spec = pl.BlockSpec((kn * b, h), lambda i: (i, 0))

    out2 = pl.pallas_call(
        functools.partial(_norm_body, float(b * l), float(l), float(b)),
        grid=(l // kn,),
        in_specs=[
            nblk_spec,
            emb_spec,
            pl.BlockSpec((kn, 1, h), lambda i: (i, 0, 0)),
            st_spec,
            emb_spec,
            st8_spec,
            row_spec,
            row_spec,
        ],
        out_specs=nblk_spec,
        out_shape=jax.ShapeDtypeStruct((l * b, h), jnp.float32),
        scratch_shapes=[pltpu.VMEM((8, h), jnp.float32)],
        compiler_params=pltpu.CompilerParams(
            dimension_semantics=("arbitrary",)
        ),
    )(x2, emb, pe3, stats8, sbh, pe_consts, w2, b2)

    return jnp.transpose(out2.reshape(l, b, h), (1, 0, 2))
